# Initial kernel scaffold; baseline (speedup 1.0000x reference)
import jax
import jax.numpy as jnp
from jax.experimental import pallas as pl


def kernel(x, edge_index, message, params):
    raise NotImplementedError("write your pallas kernel here")


# speedup vs baseline: 3.2359x; 3.2359x over previous
import dataclasses
import functools

import jax
import jax.numpy as jnp
from jax import lax
from jax.experimental import pallas as pl
from jax.experimental.pallas import tpu as pltpu
from jax.experimental.pallas import tpu_sc as plsc

_EPS = 1e-5
_NC = 2
_NS = 16
_NW = _NC * _NS
_L = 16
_K = 80


def _fold_bn(W, b, gamma, beta):
    s = gamma * (1.0 / jnp.sqrt(1.0 + _EPS))
    return W * s[None, :], b * s + beta


def _tc_pre(x, w1x, b1, w2x, b2):
    n, d = x.shape
    h = w1x.shape[1]

    def body(x_ref, w1_ref, b1_ref, w2_ref, b2_ref, p_ref, q_ref):
        xv = x_ref[...]
        p_ref[...] = jnp.dot(xv, w1_ref[...],
                             preferred_element_type=jnp.float32) + b1_ref[...]
        q_ref[...] = jnp.dot(xv, w2_ref[...],
                             preferred_element_type=jnp.float32) + b2_ref[...]

    return pl.pallas_call(
        body,
        out_shape=(jax.ShapeDtypeStruct((n, h), jnp.float32),
                   jax.ShapeDtypeStruct((n, h), jnp.float32)),
    )(x, w1x, b1.reshape(1, -1), w2x, b2.reshape(1, -1))


def _sc_compiler_params():
    cp = pltpu.CompilerParams()
    if "needs_layout_passes" in pltpu.CompilerParams.__dataclass_fields__:
        cp = dataclasses.replace(cp, needs_layout_passes=False)
    return cp


def _sc_gather_count(p_tbl, row3d, col3d, e):
    n, h = p_tbl.shape
    per_w = row3d.shape[1]
    mesh = plsc.VectorSubcoreMesh(core_axis_name="c", subcore_axis_name="s")

    @functools.partial(
        pl.kernel,
        out_type=(jax.ShapeDtypeStruct((e, h), jnp.float32),
                  jax.ShapeDtypeStruct((_NW, n), jnp.float32)),
        mesh=mesh,
        scratch_types=[
            pltpu.VMEM((per_w, _K), jnp.int32),
            pltpu.VMEM((per_w, _K), jnp.int32),
            pltpu.VMEM((_K, h), jnp.float32),
            pltpu.VMEM((n,), jnp.float32),
            pltpu.SemaphoreType.DMA,
        ],
        compiler_params=_sc_compiler_params(),
    )
    def k(p_hbm, ridx_hbm, cidx_hbm, g_hbm, c_hbm,
          ridx_v, cidx_v, rows_v, cnt_ref, sem):
        wid = lax.axis_index("s") * _NC + lax.axis_index("c")
        base = wid * per_w
        pltpu.sync_copy(ridx_hbm.at[wid], ridx_v)
        pltpu.sync_copy(cidx_hbm.at[wid], cidx_v)
        zero16 = jnp.zeros((_L,), jnp.float32)
        one16 = jnp.full((_L,), 1.0, jnp.float32)

        @pl.loop(0, n // _L)
        def _(i):
            cnt_ref[pl.ds(i * _L, _L)] = zero16

        @pl.loop(0, per_w)
        def _(j):
            cp = pltpu.async_copy(p_hbm.at[ridx_v.at[j]], rows_v, sem)

            @pl.loop(0, _K // _L)
            def _(t):
                idx16 = cidx_v[j, pl.ds(t * _L, _L)]
                plsc.addupdate_scatter(cnt_ref, [idx16], one16)

            cp.wait()
            pltpu.sync_copy(rows_v, g_hbm.at[pl.ds((base + j) * _K, _K)])

        pltpu.sync_copy(cnt_ref, c_hbm.at[wid])

    return k(p_tbl, row3d, col3d)


def _tc_edge_mlp(msg, g, w1m, w1b, b1b, tile):
    e, h = msg.shape

    def body(m_ref, g_ref, wm_ref, wb_ref, bb_ref, o_ref):
        hid = jnp.maximum(
            jnp.dot(m_ref[...], wm_ref[...],
                    preferred_element_type=jnp.float32) + g_ref[...], 0.0)
        o_ref[...] = jnp.maximum(
            jnp.dot(hid, wb_ref[...],
                    preferred_element_type=jnp.float32) + bb_ref[...], 0.0)

    return pl.pallas_call(
        body,
        grid=(e // tile,),
        in_specs=[
            pl.BlockSpec((tile, h), lambda i: (i, 0)),
            pl.BlockSpec((tile, h), lambda i: (i, 0)),
            pl.BlockSpec((h, h), lambda i: (0, 0)),
            pl.BlockSpec((h, h), lambda i: (0, 0)),
            pl.BlockSpec((1, h), lambda i: (0, 0)),
        ],
        out_specs=pl.BlockSpec((tile, h), lambda i: (i, 0)),
        out_shape=jax.ShapeDtypeStruct((e, h), jnp.float32),
    )(msg, g, w1m, w1b, b1b.reshape(1, -1))


def _sc_scatter(h2, col3d, n):
    e, h = h2.shape
    per_w = col3d.shape[1]
    mesh = plsc.VectorSubcoreMesh(core_axis_name="c", subcore_axis_name="s")
    zeros_v = jnp.zeros((n, h), jnp.float32)

    @functools.partial(
        pl.kernel,
        out_type=jax.ShapeDtypeStruct((_NC, n, h), jnp.float32),
        mesh=mesh,
        scratch_types=[
            pltpu.VMEM((per_w, _K), jnp.int32),
            pltpu.VMEM((_K, h), jnp.float32),
            pltpu.VMEM_SHARED((n, h), jnp.float32),
        ],
    )
    def k(h2_hbm, col_hbm, zv_hbm, s_hbm, idx_v, vals_v, acc_sh):
        cid = lax.axis_index("c")
        sid = lax.axis_index("s")
        wid = sid * _NC + cid
        base = wid * per_w
        pltpu.sync_copy(col_hbm.at[wid], idx_v)

        @pl.when(sid == 0)
        def _():
            pltpu.sync_copy(zv_hbm, acc_sh)

        plsc.subcore_barrier()

        @pl.loop(0, per_w)
        def _(j):
            pltpu.sync_copy(h2_hbm.at[pl.ds((base + j) * _K, _K)], vals_v)
            pltpu.sync_copy(vals_v, acc_sh.at[idx_v.at[j]], add=True)

        plsc.subcore_barrier()

        @pl.when(sid == 0)
        def _():
            pltpu.sync_copy(acc_sh, s_hbm.at[cid])

    return k(h2, col3d, zeros_v)


def _tc_node(q, s_part, cnt_t, wagg, w2b, b2b, wa, ba):
    n, h = q.shape

    def body(q_ref, s_ref, c_ref, wg_ref, wb_ref, bb_ref, wa_ref, ba_ref,
             out_ref, att_ref):
        ssum = s_ref[0] + s_ref[1]
        cnt = jnp.sum(c_ref[...], axis=1, keepdims=True)
        agg = jnp.where(cnt > 0.0, ssum / jnp.maximum(cnt, 1.0), 0.0)
        hid = jnp.maximum(
            q_ref[...] + jnp.dot(agg, wg_ref[...],
                                 preferred_element_type=jnp.float32), 0.0)
        h2 = jnp.maximum(
            jnp.dot(hid, wb_ref[...],
                    preferred_element_type=jnp.float32) + bb_ref[...], 0.0)
        out_ref[...] = h2
        logit = jnp.sum(h2 * wa_ref[...], axis=1, keepdims=True)
        att_ref[...] = jax.nn.sigmoid(logit + ba_ref[...][0:1, 0:1])

    return pl.pallas_call(
        body,
        out_shape=(jax.ShapeDtypeStruct((n, h), jnp.float32),
                   jax.ShapeDtypeStruct((n, 1), jnp.float32)),
    )(q, s_part, cnt_t, wagg, w2b, b2b.reshape(1, -1),
      wa.reshape(1, -1), ba.reshape(1, 1))


def kernel(x, edge_index, message, params):
    n, in_dim = x.shape
    e = edge_index.shape[1]
    h = message.shape[1]
    assert e % (_NW * _K) == 0 and n % 8 == 0

    w1a, b1a = _fold_bn(params['W1a'], params['b1a'],
                        params['g1a'], params['be1a'])
    w1b, b1b = _fold_bn(params['W1b'], params['b1b'],
                        params['g1b'], params['be1b'])
    w2a, b2a = _fold_bn(params['W2a'], params['b2a'],
                        params['g2a'], params['be2a'])
    w2b, b2b = _fold_bn(params['W2b'], params['b2b'],
                        params['g2b'], params['be2b'])
    w1m, w1x = w1a[:h], w1a[h:]
    w2x, wagg = w2a[:in_dim], w2a[in_dim:]

    row3d = edge_index[0].reshape(_NW, -1, _K)
    col3d = edge_index[1].reshape(_NW, -1, _K)

    p_tbl, q_tbl = _tc_pre(x, w1x, b1a, w2x, b2a)
    g, cnt = _sc_gather_count(p_tbl, row3d, col3d, e)
    h2 = _tc_edge_mlp(message, g, w1m, w1b, b1b, tile=2000)
    s_part = _sc_scatter(h2, col3d, n)
    out, att = _tc_node(q_tbl, s_part, cnt.T, wagg, w2b, b2b,
                        params['Wa'], params['ba'])
    return (out, att.reshape(-1))


# speedup vs baseline: 4.2392x; 1.3100x over previous
import dataclasses
import functools

import jax
import jax.numpy as jnp
from jax import lax
from jax.experimental import pallas as pl
from jax.experimental.pallas import tpu as pltpu
from jax.experimental.pallas import tpu_sc as plsc

_EPS = 1e-5
_NC = 2
_NS = 16
_NW = _NC * _NS
_L = 16
_K = 80


def _fold_bn(W, b, gamma, beta):
    s = gamma * (1.0 / jnp.sqrt(1.0 + _EPS))
    return W * s[None, :], b * s + beta


def _tc_pre(x, w1x, b1, w2x, b2):
    n, d = x.shape
    h = w1x.shape[1]

    def body(x_ref, w1_ref, b1_ref, w2_ref, b2_ref, p_ref, q_ref):
        xv = x_ref[...]
        p_ref[...] = jnp.dot(xv, w1_ref[...],
                             preferred_element_type=jnp.float32) + b1_ref[...]
        q_ref[...] = jnp.dot(xv, w2_ref[...],
                             preferred_element_type=jnp.float32) + b2_ref[...]

    return pl.pallas_call(
        body,
        out_shape=(jax.ShapeDtypeStruct((n, h), jnp.float32),
                   jax.ShapeDtypeStruct((n, h), jnp.float32)),
    )(x, w1x, b1.reshape(1, -1), w2x, b2.reshape(1, -1))


def _sc_compiler_params():
    cp = pltpu.CompilerParams()
    if "needs_layout_passes" in pltpu.CompilerParams.__dataclass_fields__:
        cp = dataclasses.replace(cp, needs_layout_passes=False)
    return cp


def _sc_gather_count(p_tbl, row3d, col3d, e):
    n, h = p_tbl.shape
    per_w = row3d.shape[1]
    mesh = plsc.VectorSubcoreMesh(core_axis_name="c", subcore_axis_name="s")

    @functools.partial(
        pl.kernel,
        out_type=(jax.ShapeDtypeStruct((e, h), jnp.float32),
                  jax.ShapeDtypeStruct((_NW, n), jnp.float32)),
        mesh=mesh,
        scratch_types=[
            pltpu.VMEM((per_w, _K), jnp.int32),
            pltpu.VMEM((per_w, _K), jnp.int32),
            pltpu.VMEM((_K, h), jnp.float32),
            pltpu.VMEM((_K, h), jnp.float32),
            pltpu.VMEM((n,), jnp.float32),
            pltpu.SemaphoreType.DMA,
            pltpu.SemaphoreType.DMA,
        ],
        compiler_params=_sc_compiler_params(),
    )
    def k(p_hbm, ridx_hbm, cidx_hbm, g_hbm, c_hbm,
          ridx_v, cidx_v, rows0, rows1, cnt_ref, sem0, sem1):
        wid = lax.axis_index("s") * _NC + lax.axis_index("c")
        base = wid * per_w
        pltpu.sync_copy(ridx_hbm.at[wid], ridx_v)
        pltpu.sync_copy(cidx_hbm.at[wid], cidx_v)
        zero16 = jnp.zeros((_L,), jnp.float32)
        one16 = jnp.full((_L,), 1.0, jnp.float32)

        @pl.loop(0, n // _L)
        def _(i):
            cnt_ref[pl.ds(i * _L, _L)] = zero16

        def hist(j):
            @pl.loop(0, _K // _L)
            def _(t):
                idx16 = cidx_v[j, pl.ds(t * _L, _L)]
                plsc.addupdate_scatter(cnt_ref, [idx16], one16)

        def wr(j, buf):
            pltpu.sync_copy(buf, g_hbm.at[pl.ds((base + j) * _K, _K)])

        pltpu.async_copy(p_hbm.at[ridx_v.at[0]], rows0, sem0)

        @pl.loop(0, (per_w - 1) // 2)
        def _(p):
            j = 2 * p
            d1 = pltpu.async_copy(p_hbm.at[ridx_v.at[j + 1]], rows1, sem1)
            hist(j)
            pltpu.make_async_copy(p_hbm.at[ridx_v.at[j]], rows0, sem0).wait()
            wr(j, rows0)
            pltpu.async_copy(p_hbm.at[ridx_v.at[j + 2]], rows0, sem0)
            hist(j + 1)
            d1.wait()
            wr(j + 1, rows1)

        hist(per_w - 1)
        pltpu.make_async_copy(p_hbm.at[ridx_v.at[0]], rows0, sem0).wait()
        wr(per_w - 1, rows0)
        pltpu.sync_copy(cnt_ref, c_hbm.at[wid])

    return k(p_tbl, row3d, col3d)


def _tc_edge_mlp(msg, g, w1m, w1b, b1b, tile):
    e, h = msg.shape

    def body(m_ref, g_ref, wm_ref, wb_ref, bb_ref, o_ref):
        hid = jnp.maximum(
            jnp.dot(m_ref[...], wm_ref[...],
                    preferred_element_type=jnp.float32) + g_ref[...], 0.0)
        o_ref[...] = jnp.maximum(
            jnp.dot(hid, wb_ref[...],
                    preferred_element_type=jnp.float32) + bb_ref[...], 0.0)

    return pl.pallas_call(
        body,
        grid=(e // tile,),
        in_specs=[
            pl.BlockSpec((tile, h), lambda i: (i, 0)),
            pl.BlockSpec((tile, h), lambda i: (i, 0)),
            pl.BlockSpec((h, h), lambda i: (0, 0)),
            pl.BlockSpec((h, h), lambda i: (0, 0)),
            pl.BlockSpec((1, h), lambda i: (0, 0)),
        ],
        out_specs=pl.BlockSpec((tile, h), lambda i: (i, 0)),
        out_shape=jax.ShapeDtypeStruct((e, h), jnp.float32),
    )(msg, g, w1m, w1b, b1b.reshape(1, -1))


def _sc_scatter(h2, col3d, n):
    e, h = h2.shape
    per_w = col3d.shape[1]
    mesh = plsc.VectorSubcoreMesh(core_axis_name="c", subcore_axis_name="s")
    zeros_v = jnp.zeros((n, h), jnp.float32)

    @functools.partial(
        pl.kernel,
        out_type=jax.ShapeDtypeStruct((_NC, n, h), jnp.float32),
        mesh=mesh,
        scratch_types=[
            pltpu.VMEM((per_w, _K), jnp.int32),
            pltpu.VMEM((_K, h), jnp.float32),
            pltpu.VMEM((_K, h), jnp.float32),
            pltpu.VMEM_SHARED((n, h), jnp.float32),
            pltpu.SemaphoreType.DMA,
            pltpu.SemaphoreType.DMA,
        ],
    )
    def k(h2_hbm, col_hbm, zv_hbm, s_hbm, idx_v, b0, b1, acc_sh, sl0, sl1):
        cid = lax.axis_index("c")
        sid = lax.axis_index("s")
        wid = sid * _NC + cid
        base = wid * per_w
        pltpu.sync_copy(col_hbm.at[wid], idx_v)

        @pl.when(sid == 0)
        def _():
            pltpu.sync_copy(zv_hbm, acc_sh)

        plsc.subcore_barrier()

        def load(i, buf, sem):
            return pltpu.async_copy(
                h2_hbm.at[pl.ds((base + i) * _K, _K)], buf, sem)

        def scat(i, buf):
            pltpu.sync_copy(buf, acc_sh.at[idx_v.at[i]], add=True)

        load(0, b0, sl0)

        @pl.loop(0, (per_w - 1) // 2)
        def _(p):
            i = 2 * p
            d1 = load(i + 1, b1, sl1)
            pltpu.make_async_copy(h2_hbm.at[pl.ds(0, _K)], b0, sl0).wait()
            scat(i, b0)
            load(i + 2, b0, sl0)
            d1.wait()
            scat(i + 1, b1)

        pltpu.make_async_copy(h2_hbm.at[pl.ds(0, _K)], b0, sl0).wait()
        scat(per_w - 1, b0)

        plsc.subcore_barrier()

        @pl.when(sid == 0)
        def _():
            pltpu.sync_copy(acc_sh, s_hbm.at[cid])

    return k(h2, col3d, zeros_v)


def _tc_node(q, s_part, cnt_t, wagg, w2b, b2b, wa, ba):
    n, h = q.shape

    def body(q_ref, s_ref, c_ref, wg_ref, wb_ref, bb_ref, wa_ref, ba_ref,
             out_ref, att_ref):
        ssum = s_ref[0] + s_ref[1]
        cnt = jnp.sum(c_ref[...], axis=1, keepdims=True)
        agg = jnp.where(cnt > 0.0, ssum / jnp.maximum(cnt, 1.0), 0.0)
        hid = jnp.maximum(
            q_ref[...] + jnp.dot(agg, wg_ref[...],
                                 preferred_element_type=jnp.float32), 0.0)
        h2 = jnp.maximum(
            jnp.dot(hid, wb_ref[...],
                    preferred_element_type=jnp.float32) + bb_ref[...], 0.0)
        out_ref[...] = h2
        logit = jnp.sum(h2 * wa_ref[...], axis=1, keepdims=True)
        att_ref[...] = jax.nn.sigmoid(logit + ba_ref[...][0:1, 0:1])

    return pl.pallas_call(
        body,
        out_shape=(jax.ShapeDtypeStruct((n, h), jnp.float32),
                   jax.ShapeDtypeStruct((n, 1), jnp.float32)),
    )(q, s_part, cnt_t, wagg, w2b, b2b.reshape(1, -1),
      wa.reshape(1, -1), ba.reshape(1, 1))


def kernel(x, edge_index, message, params):
    n, in_dim = x.shape
    e = edge_index.shape[1]
    h = message.shape[1]
    assert e % (_NW * _K) == 0 and n % 8 == 0

    w1a, b1a = _fold_bn(params['W1a'], params['b1a'],
                        params['g1a'], params['be1a'])
    w1b, b1b = _fold_bn(params['W1b'], params['b1b'],
                        params['g1b'], params['be1b'])
    w2a, b2a = _fold_bn(params['W2a'], params['b2a'],
                        params['g2a'], params['be2a'])
    w2b, b2b = _fold_bn(params['W2b'], params['b2b'],
                        params['g2b'], params['be2b'])
    w1m, w1x = w1a[:h], w1a[h:]
    w2x, wagg = w2a[:in_dim], w2a[in_dim:]

    row3d = edge_index[0].reshape(_NW, -1, _K)
    col3d = edge_index[1].reshape(_NW, -1, _K)

    p_tbl, q_tbl = _tc_pre(x, w1x, b1a, w2x, b2a)
    g, cnt = _sc_gather_count(p_tbl, row3d, col3d, e)
    h2 = _tc_edge_mlp(message, g, w1m, w1b, b1b, tile=2000)
    s_part = _sc_scatter(h2, col3d, n)
    out, att = _tc_node(q_tbl, s_part, cnt.T, wagg, w2b, b2b,
                        params['Wa'], params['ba'])
    return (out, att.reshape(-1))


# speedup vs baseline: 4.7171x; 1.1127x over previous
import dataclasses
import functools

import jax
import jax.numpy as jnp
from jax import lax
from jax.experimental import pallas as pl
from jax.experimental.pallas import tpu as pltpu
from jax.experimental.pallas import tpu_sc as plsc

_EPS = 1e-5
_NC = 2
_NS = 16
_NW = _NC * _NS
_L = 16
_K = 80


def _fold_bn(W, b, gamma, beta):
    s = gamma * (1.0 / jnp.sqrt(1.0 + _EPS))
    return W * s[None, :], b * s + beta


def _tc_pre(x, w1x, b1, w2x, b2):
    n, d = x.shape
    h = w1x.shape[1]

    def body(x_ref, w1_ref, b1_ref, w2_ref, b2_ref, p_ref, q_ref):
        xv = x_ref[...]
        p_ref[...] = jnp.dot(xv, w1_ref[...],
                             preferred_element_type=jnp.float32) + b1_ref[...]
        q_ref[...] = jnp.dot(xv, w2_ref[...],
                             preferred_element_type=jnp.float32) + b2_ref[...]

    return pl.pallas_call(
        body,
        out_shape=(jax.ShapeDtypeStruct((n, h), jnp.float32),
                   jax.ShapeDtypeStruct((n, h), jnp.float32)),
    )(x, w1x, b1.reshape(1, -1), w2x, b2.reshape(1, -1))


def _sc_compiler_params():
    cp = pltpu.CompilerParams()
    if "needs_layout_passes" in pltpu.CompilerParams.__dataclass_fields__:
        cp = dataclasses.replace(cp, needs_layout_passes=False)
    return cp


def _sc_gather(p_tbl, row3d, col3d, n_hist):
    h = p_tbl.shape[1]
    per_w = row3d.shape[1]
    e = _NW * per_w * _K
    last = per_w - 1
    with_hist = col3d is not None
    mesh = plsc.VectorSubcoreMesh(core_axis_name="c", subcore_axis_name="s")

    out_type = [jax.ShapeDtypeStruct((e, h), jnp.float32)]
    scratch = [
        pltpu.VMEM((per_w, _K), jnp.int32),
        pltpu.VMEM((_K, h), jnp.float32),
        pltpu.VMEM((_K, h), jnp.float32),
        pltpu.SemaphoreType.DMA,
        pltpu.SemaphoreType.DMA,
    ]
    if with_hist:
        out_type.append(jax.ShapeDtypeStruct((_NW, n_hist), jnp.float32))
        scratch += [pltpu.VMEM((col3d.shape[1], _K), jnp.int32),
                    pltpu.VMEM((n_hist,), jnp.float32)]

    @functools.partial(
        pl.kernel,
        out_type=tuple(out_type),
        mesh=mesh,
        scratch_types=scratch,
        compiler_params=_sc_compiler_params(),
    )
    def k(p_hbm, ridx_hbm, *args):
        if with_hist:
            cidx_hbm, g_hbm, c_hbm, ridx_v, rows0, rows1, sem0, sem1, \
                cidx_v, cnt_ref = args
        else:
            g_hbm, ridx_v, rows0, rows1, sem0, sem1 = args
        wid = lax.axis_index("s") * _NC + lax.axis_index("c")
        base = wid * per_w
        pltpu.sync_copy(ridx_hbm.at[wid], ridx_v)

        if with_hist:
            pltpu.sync_copy(cidx_hbm.at[wid], cidx_v)
            zero16 = jnp.zeros((_L,), jnp.float32)
            one16 = jnp.full((_L,), 1.0, jnp.float32)

            @pl.loop(0, n_hist // _L)
            def _(i):
                cnt_ref[pl.ds(i * _L, _L)] = zero16

            @pl.loop(0, col3d.shape[1])
            def _(j):
                @pl.loop(0, _K // _L)
                def _(t):
                    idx16 = cidx_v[j, pl.ds(t * _L, _L)]
                    plsc.addupdate_scatter(cnt_ref, [idx16], one16)

        def g(j, buf, sem):
            pltpu.async_copy(p_hbm.at[ridx_v.at[j]], buf, sem)

        def wt(buf, sem):
            pltpu.make_async_copy(p_hbm.at[ridx_v.at[0]], buf, sem).wait()

        def wr(j, buf):
            pltpu.sync_copy(buf, g_hbm.at[pl.ds((base + j) * _K, _K)])

        g(0, rows0, sem0)

        @pl.loop(0, per_w // 2)
        def _(p):
            j = 2 * p
            g(j + 1, rows1, sem1)
            wt(rows0, sem0)
            wr(j, rows0)
            g(jnp.minimum(j + 2, last), rows0, sem0)
            wt(rows1, sem1)
            wr(j + 1, rows1)

        wt(rows0, sem0)
        if per_w % 2 == 1:
            wr(last, rows0)
        if with_hist:
            pltpu.sync_copy(cnt_ref, c_hbm.at[wid])

    if with_hist:
        return k(p_tbl, row3d, col3d)
    res = k(p_tbl, row3d)
    return res if isinstance(res, (tuple, list)) else (res,)


def _tc_edge_mlp(msg, g, w1m, w1b, b1b, tile, goff):
    h = msg.shape[1]
    sz = g.shape[0]

    def body(m_ref, g_ref, wm_ref, wb_ref, bb_ref, o_ref):
        hid = jnp.maximum(
            jnp.dot(m_ref[...], wm_ref[...],
                    preferred_element_type=jnp.float32) + g_ref[...], 0.0)
        o_ref[...] = jnp.maximum(
            jnp.dot(hid, wb_ref[...],
                    preferred_element_type=jnp.float32) + bb_ref[...], 0.0)

    return pl.pallas_call(
        body,
        grid=(sz // tile,),
        in_specs=[
            pl.BlockSpec((tile, h), lambda i: (i + goff, 0)),
            pl.BlockSpec((tile, h), lambda i: (i, 0)),
            pl.BlockSpec((h, h), lambda i: (0, 0)),
            pl.BlockSpec((h, h), lambda i: (0, 0)),
            pl.BlockSpec((1, h), lambda i: (0, 0)),
        ],
        out_specs=pl.BlockSpec((tile, h), lambda i: (i, 0)),
        out_shape=jax.ShapeDtypeStruct((sz, h), jnp.float32),
    )(msg, g, w1m, w1b, b1b.reshape(1, -1))


def _sc_scatter(h2, col3d, n):
    e, h = h2.shape
    per_w = col3d.shape[1]
    mesh = plsc.VectorSubcoreMesh(core_axis_name="c", subcore_axis_name="s")
    zeros_v = jnp.zeros((n, h), jnp.float32)

    @functools.partial(
        pl.kernel,
        out_type=jax.ShapeDtypeStruct((_NC, n, h), jnp.float32),
        mesh=mesh,
        scratch_types=[
            pltpu.VMEM((per_w, _K), jnp.int32),
            pltpu.VMEM((_K, h), jnp.float32),
            pltpu.VMEM((_K, h), jnp.float32),
            pltpu.VMEM_SHARED((n, h), jnp.float32),
            pltpu.SemaphoreType.DMA,
            pltpu.SemaphoreType.DMA,
        ],
    )
    def k(h2_hbm, col_hbm, zv_hbm, s_hbm, idx_v, b0, b1, acc_sh, sl0, sl1):
        cid = lax.axis_index("c")
        sid = lax.axis_index("s")
        wid = sid * _NC + cid
        base = wid * per_w
        pltpu.sync_copy(col_hbm.at[wid], idx_v)

        @pl.when(sid == 0)
        def _():
            pltpu.sync_copy(zv_hbm, acc_sh)

        plsc.subcore_barrier()

        def load(i, buf, sem):
            pltpu.async_copy(h2_hbm.at[pl.ds((base + i) * _K, _K)], buf, sem)

        def wt(buf, sem):
            pltpu.make_async_copy(h2_hbm.at[pl.ds(0, _K)], buf, sem).wait()

        def scat(i, buf):
            pltpu.sync_copy(buf, acc_sh.at[idx_v.at[i]], add=True)

        last = per_w - 1
        load(0, b0, sl0)

        @pl.loop(0, per_w // 2)
        def _(p):
            i = 2 * p
            load(i + 1, b1, sl1)
            wt(b0, sl0)
            scat(i, b0)
            load(jnp.minimum(i + 2, last), b0, sl0)
            wt(b1, sl1)
            scat(i + 1, b1)

        wt(b0, sl0)
        if per_w % 2 == 1:
            scat(last, b0)

        plsc.subcore_barrier()

        @pl.when(sid == 0)
        def _():
            pltpu.sync_copy(acc_sh, s_hbm.at[cid])

    return k(h2, col3d, zeros_v)


def _tc_node(q, s_part0, s_part1, cnt_t, wagg, w2b, b2b, wa, ba):
    n, h = q.shape

    def body(q_ref, s_ref, s1_ref, c_ref, wg_ref, wb_ref, bb_ref, wa_ref,
             ba_ref, out_ref, att_ref):
        ssum = (s_ref[0] + s_ref[1]) + (s1_ref[0] + s1_ref[1])
        cnt = jnp.sum(c_ref[...], axis=1, keepdims=True)
        agg = jnp.where(cnt > 0.0, ssum / jnp.maximum(cnt, 1.0), 0.0)
        hid = jnp.maximum(
            q_ref[...] + jnp.dot(agg, wg_ref[...],
                                 preferred_element_type=jnp.float32), 0.0)
        h2 = jnp.maximum(
            jnp.dot(hid, wb_ref[...],
                    preferred_element_type=jnp.float32) + bb_ref[...], 0.0)
        out_ref[...] = h2
        logit = jnp.sum(h2 * wa_ref[...], axis=1, keepdims=True)
        att_ref[...] = jax.nn.sigmoid(logit + ba_ref[...][0:1, 0:1])

    return pl.pallas_call(
        body,
        out_shape=(jax.ShapeDtypeStruct((n, h), jnp.float32),
                   jax.ShapeDtypeStruct((n, 1), jnp.float32)),
    )(q, s_part0, s_part1, cnt_t, wagg, w2b, b2b.reshape(1, -1),
      wa.reshape(1, -1), ba.reshape(1, 1))


def kernel(x, edge_index, message, params):
    n, in_dim = x.shape
    e = edge_index.shape[1]
    h = message.shape[1]
    assert e % (_NW * _K) == 0 and n % 8 == 0

    w1a, b1a = _fold_bn(params['W1a'], params['b1a'],
                        params['g1a'], params['be1a'])
    w1b, b1b = _fold_bn(params['W1b'], params['b1b'],
                        params['g1b'], params['be1b'])
    w2a, b2a = _fold_bn(params['W2a'], params['b2a'],
                        params['g2a'], params['be2a'])
    w2b, b2b = _fold_bn(params['W2b'], params['b2b'],
                        params['g2b'], params['be2b'])
    w1m, w1x = w1a[:h], w1a[h:]
    w2x, wagg = w2a[:in_dim], w2a[in_dim:]

    tile = _NW * _K
    grp_total = e // tile
    grp0 = (grp_total + 1) // 2
    sz0 = grp0 * tile
    row, col = edge_index[0], edge_index[1]
    col3d_full = col.reshape(_NW, -1, _K)

    p_tbl, q_tbl = _tc_pre(x, w1x, b1a, w2x, b2a)

    s_parts = []
    cnt = None
    for c, (off, sz) in enumerate(((0, sz0), (sz0, e - sz0))):
        row3d = lax.dynamic_slice_in_dim(row, off, sz).reshape(_NW, -1, _K)
        col3d = lax.dynamic_slice_in_dim(col, off, sz).reshape(_NW, -1, _K)
        if c == 0:
            g, cnt = _sc_gather(p_tbl, row3d, col3d_full, n)
        else:
            (g,) = _sc_gather(p_tbl, row3d, None, n)
        h2 = _tc_edge_mlp(message, g, w1m, w1b, b1b, tile, off // tile)
        s_parts.append(_sc_scatter(h2, col3d, n))

    out, att = _tc_node(q_tbl, s_parts[0], s_parts[1], cnt.T, wagg, w2b,
                        b2b, params['Wa'], params['ba'])
    return (out, att.reshape(-1))
